# Initial kernel scaffold; baseline (speedup 1.0000x reference)
import jax
import jax.numpy as jnp
from jax.experimental import pallas as pl


def kernel(x, panel_final, idx_xi, idx_eta, tx, ty):
    raise NotImplementedError("write your pallas kernel here")


# speedup vs baseline: 1.1597x; 1.1597x over previous
import functools

import jax
import jax.numpy as jnp
from jax import lax
from jax.experimental import pallas as pl
from jax.experimental.pallas import tpu as pltpu
from jax.experimental.pallas import tpu_sc as plsc

NUM_PANEL = 6
NUM_ELEM = 224
PAD = 3
NUM_DIR = 4
N_WITH_PAD = NUM_ELEM + 2 * PAD
N_FEAT = 128
N_BATCH = 2

N_POINTS = NUM_PANEL * NUM_DIR * PAD * N_WITH_PAD
N_ROWS = 4 * N_BATCH * N_POINTS
CHUNK = 256


def _sc_gather(table, idx, n_workers, n_chunks):
    b_per_w = n_chunks * CHUNK
    mesh = plsc.VectorSubcoreMesh(core_axis_name="c", subcore_axis_name="s")
    info = plsc.get_sparse_core_info()
    nc = info.num_cores

    @functools.partial(
        pl.kernel,
        mesh=mesh,
        out_type=jax.ShapeDtypeStruct((n_workers * b_per_w, N_FEAT), jnp.float32),
        scratch_types=[
            pltpu.VMEM((CHUNK,), jnp.int32),
            pltpu.VMEM((CHUNK, N_FEAT), jnp.float32),
            pltpu.SemaphoreType.DMA,
        ],
    )
    def k(table_hbm, idx_hbm, out_hbm, idx_v, rows_v, sem):
        wid = lax.axis_index("s") * nc + lax.axis_index("c")
        base = wid * b_per_w
        for c in range(n_chunks):
            off = base + c * CHUNK
            pltpu.sync_copy(idx_hbm.at[pl.ds(off, CHUNK)], idx_v)
            pltpu.async_copy(table_hbm.at[idx_v], rows_v, sem).wait()
            pltpu.sync_copy(rows_v, out_hbm.at[pl.ds(off, CHUNK)])

    return k(table, idx)


def _assemble_kernel(x_ref, g_ref, tx_ref, ty_ref, o_ref):
    x = x_ref[0, 0]
    g = g_ref[:, 0, 0]
    q11 = g[0]
    q12 = g[1]
    q21 = g[2]
    q22 = g[3]
    tx = tx_ref[0]
    ty = ty_ref[0]
    r1 = q11 + tx * (q21 - q11)
    r2 = q12 + tx * (q22 - q12)
    iv = r1 + ty * (r2 - r1)
    iv0 = iv[:, 0]
    iv1 = iv[:, 1]
    iv2t = jnp.swapaxes(iv[:, 2], -1, -2)
    iv3t = jnp.swapaxes(iv[:, 3], -1, -2)
    top = jnp.concatenate(
        [iv2t[:, :PAD, :], iv0[:, :, PAD:-PAD], iv3t[:, :PAD, :]], axis=-1
    )
    bot = jnp.concatenate(
        [iv2t[:, -PAD:, :], iv1[:, :, PAD:-PAD], iv3t[:, -PAD:, :]], axis=-1
    )
    mid = jnp.concatenate([iv2t[:, PAD:-PAD, :], x, iv3t[:, PAD:-PAD, :]], axis=-1)
    o_ref[0, 0] = jnp.concatenate([top, mid, bot], axis=-2)


def _assemble(xr, gm, tx, ty, f_blk=32):
    nf = N_FEAT // f_blk
    return pl.pallas_call(
        _assemble_kernel,
        grid=(N_BATCH, NUM_PANEL, nf),
        in_specs=[
            pl.BlockSpec(
                (1, 1, f_blk, NUM_ELEM, NUM_ELEM), lambda b, s, f: (b, s, f, 0, 0)
            ),
            pl.BlockSpec(
                (4, 1, 1, f_blk, NUM_DIR, PAD, N_WITH_PAD),
                lambda b, s, f: (0, b, s, f, 0, 0, 0),
            ),
            pl.BlockSpec((1, NUM_DIR, PAD, N_WITH_PAD), lambda b, s, f: (s, 0, 0, 0)),
            pl.BlockSpec((1, NUM_DIR, PAD, N_WITH_PAD), lambda b, s, f: (s, 0, 0, 0)),
        ],
        out_specs=pl.BlockSpec(
            (1, 1, f_blk, N_WITH_PAD, N_WITH_PAD), lambda b, s, f: (b, s, f, 0, 0)
        ),
        out_shape=jax.ShapeDtypeStruct(
            (N_BATCH, NUM_PANEL, N_FEAT, N_WITH_PAD, N_WITH_PAD), jnp.float32
        ),
    )(xr, gm, tx, ty)


def kernel(x, panel_final, idx_xi, idx_eta, tx, ty):
    xr = x.reshape(N_BATCH, NUM_PANEL, N_FEAT, NUM_ELEM, NUM_ELEM)
    table = jnp.transpose(xr, (0, 1, 3, 4, 2)).reshape(-1, N_FEAT)

    pf = panel_final.astype(jnp.int32).reshape(NUM_PANEL, NUM_DIR, PAD, N_WITH_PAD)
    xi = idx_xi.astype(jnp.int32).reshape(NUM_PANEL, NUM_DIR, PAD, N_WITH_PAD)
    eta = idx_eta.astype(jnp.int32).reshape(NUM_PANEL, NUM_DIR, PAD, N_WITH_PAD)
    txr = tx.astype(jnp.float32).reshape(NUM_PANEL, NUM_DIR, PAD, N_WITH_PAD)
    tyr = ty.astype(jnp.float32).reshape(NUM_PANEL, NUM_DIR, PAD, N_WITH_PAD)

    b_ids = jnp.arange(N_BATCH, dtype=jnp.int32).reshape(N_BATCH, 1, 1, 1, 1)
    r00 = ((b_ids * NUM_PANEL + pf) * NUM_ELEM + xi) * NUM_ELEM + eta
    idx = jnp.stack([r00, r00 + NUM_ELEM, r00 + 1, r00 + NUM_ELEM + 1], axis=1)
    idx = idx.reshape(N_ROWS)

    info = plsc.get_sparse_core_info()
    n_workers = info.num_cores * info.num_subcores
    per_w = -(-N_ROWS // (n_workers * CHUNK)) * CHUNK
    n_pad = n_workers * per_w
    idx = jnp.concatenate([idx, jnp.zeros(n_pad - N_ROWS, dtype=jnp.int32)])

    g = _sc_gather(table, idx, n_workers, per_w // CHUNK)

    g7 = g[:N_ROWS].reshape(N_BATCH, 4, NUM_PANEL, NUM_DIR, PAD, N_WITH_PAD, N_FEAT)
    gm = jnp.transpose(g7, (1, 0, 2, 6, 3, 4, 5))

    out = _assemble(xr, gm, txr, tyr)
    return out.reshape(N_BATCH * NUM_PANEL, N_FEAT, N_WITH_PAD, N_WITH_PAD)


# speedup vs baseline: 1.1813x; 1.0186x over previous
import functools

import jax
import jax.numpy as jnp
from jax import lax
from jax.experimental import pallas as pl
from jax.experimental.pallas import tpu as pltpu
from jax.experimental.pallas import tpu_sc as plsc

NUM_PANEL = 6
NUM_ELEM = 224
PAD = 3
NUM_DIR = 4
N_WITH_PAD = NUM_ELEM + 2 * PAD
N_FEAT = 128
N_BATCH = 2

N_POINTS = NUM_PANEL * NUM_DIR * PAD * N_WITH_PAD
N_ROWS = 4 * N_BATCH * N_POINTS
CHUNK = 256


def _sc_gather(table, idx, n_workers, n_chunks):
    b_per_w = n_chunks * CHUNK
    mesh = plsc.VectorSubcoreMesh(core_axis_name="c", subcore_axis_name="s")
    info = plsc.get_sparse_core_info()
    nc = info.num_cores

    @functools.partial(
        pl.kernel,
        mesh=mesh,
        out_type=jax.ShapeDtypeStruct((n_workers * b_per_w, N_FEAT), jnp.float32),
        scratch_types=[
            pltpu.VMEM((CHUNK,), jnp.int32),
            pltpu.VMEM((CHUNK,), jnp.int32),
            pltpu.VMEM((CHUNK, N_FEAT), jnp.float32),
            pltpu.VMEM((CHUNK, N_FEAT), jnp.float32),
            pltpu.SemaphoreType.DMA,
            pltpu.SemaphoreType.DMA,
        ],
    )
    def k(table_hbm, idx_hbm, out_hbm, idx_v0, idx_v1, rows_v0, rows_v1, sem0, sem1):
        wid = lax.axis_index("s") * nc + lax.axis_index("c")
        base = wid * b_per_w
        bufs = [(idx_v0, rows_v0, sem0), (idx_v1, rows_v1, sem1)]
        pending = []
        for c in range(n_chunks):
            idx_v, rows_v, sem = bufs[c % 2]
            off = base + c * CHUNK
            pltpu.sync_copy(idx_hbm.at[pl.ds(off, CHUNK)], idx_v)
            cp = pltpu.async_copy(table_hbm.at[idx_v], rows_v, sem)
            pending.append((cp, rows_v, off))
            if c >= 1:
                pcp, prows, poff = pending[c - 1]
                pcp.wait()
                pltpu.sync_copy(prows, out_hbm.at[pl.ds(poff, CHUNK)])
        lcp, lrows, loff = pending[n_chunks - 1]
        lcp.wait()
        pltpu.sync_copy(lrows, out_hbm.at[pl.ds(loff, CHUNK)])

    return k(table, idx)


def _assemble_kernel(x_ref, g_ref, tx_ref, ty_ref, o_ref):
    x = x_ref[0, 0]
    g = g_ref[:, 0, 0]
    q11 = g[0]
    q12 = g[1]
    q21 = g[2]
    q22 = g[3]
    tx = tx_ref[0]
    ty = ty_ref[0]
    r1 = q11 + tx * (q21 - q11)
    r2 = q12 + tx * (q22 - q12)
    iv = r1 + ty * (r2 - r1)
    iv0 = iv[:, 0]
    iv1 = iv[:, 1]
    iv2t = jnp.swapaxes(iv[:, 2], -1, -2)
    iv3t = jnp.swapaxes(iv[:, 3], -1, -2)
    top = jnp.concatenate(
        [iv2t[:, :PAD, :], iv0[:, :, PAD:-PAD], iv3t[:, :PAD, :]], axis=-1
    )
    bot = jnp.concatenate(
        [iv2t[:, -PAD:, :], iv1[:, :, PAD:-PAD], iv3t[:, -PAD:, :]], axis=-1
    )
    mid = jnp.concatenate([iv2t[:, PAD:-PAD, :], x, iv3t[:, PAD:-PAD, :]], axis=-1)
    o_ref[0, 0] = jnp.concatenate([top, mid, bot], axis=-2)


def _assemble(xr, gm, tx, ty, f_blk=32):
    nf = N_FEAT // f_blk
    return pl.pallas_call(
        _assemble_kernel,
        grid=(N_BATCH, NUM_PANEL, nf),
        in_specs=[
            pl.BlockSpec(
                (1, 1, f_blk, NUM_ELEM, NUM_ELEM), lambda b, s, f: (b, s, f, 0, 0)
            ),
            pl.BlockSpec(
                (4, 1, 1, f_blk, NUM_DIR, PAD, N_WITH_PAD),
                lambda b, s, f: (0, b, s, f, 0, 0, 0),
            ),
            pl.BlockSpec((1, NUM_DIR, PAD, N_WITH_PAD), lambda b, s, f: (s, 0, 0, 0)),
            pl.BlockSpec((1, NUM_DIR, PAD, N_WITH_PAD), lambda b, s, f: (s, 0, 0, 0)),
        ],
        out_specs=pl.BlockSpec(
            (1, 1, f_blk, N_WITH_PAD, N_WITH_PAD), lambda b, s, f: (b, s, f, 0, 0)
        ),
        out_shape=jax.ShapeDtypeStruct(
            (N_BATCH, NUM_PANEL, N_FEAT, N_WITH_PAD, N_WITH_PAD), jnp.float32
        ),
    )(xr, gm, tx, ty)


def kernel(x, panel_final, idx_xi, idx_eta, tx, ty):
    xr = x.reshape(N_BATCH, NUM_PANEL, N_FEAT, NUM_ELEM, NUM_ELEM)
    table = jnp.transpose(xr, (0, 1, 3, 4, 2)).reshape(-1, N_FEAT)

    pf = panel_final.astype(jnp.int32).reshape(NUM_PANEL, NUM_DIR, PAD, N_WITH_PAD)
    xi = idx_xi.astype(jnp.int32).reshape(NUM_PANEL, NUM_DIR, PAD, N_WITH_PAD)
    eta = idx_eta.astype(jnp.int32).reshape(NUM_PANEL, NUM_DIR, PAD, N_WITH_PAD)
    txr = tx.astype(jnp.float32).reshape(NUM_PANEL, NUM_DIR, PAD, N_WITH_PAD)
    tyr = ty.astype(jnp.float32).reshape(NUM_PANEL, NUM_DIR, PAD, N_WITH_PAD)

    b_ids = jnp.arange(N_BATCH, dtype=jnp.int32).reshape(N_BATCH, 1, 1, 1, 1)
    r00 = ((b_ids * NUM_PANEL + pf) * NUM_ELEM + xi) * NUM_ELEM + eta
    idx = jnp.stack([r00, r00 + NUM_ELEM, r00 + 1, r00 + NUM_ELEM + 1], axis=1)
    idx = idx.reshape(N_ROWS)

    info = plsc.get_sparse_core_info()
    n_workers = info.num_cores * info.num_subcores
    per_w = -(-N_ROWS // (n_workers * CHUNK)) * CHUNK
    n_pad = n_workers * per_w
    idx = jnp.concatenate([idx, jnp.zeros(n_pad - N_ROWS, dtype=jnp.int32)])

    g = _sc_gather(table, idx, n_workers, per_w // CHUNK)

    g7 = g[:N_ROWS].reshape(N_BATCH, 4, NUM_PANEL, NUM_DIR, PAD, N_WITH_PAD, N_FEAT)
    gm = jnp.transpose(g7, (1, 0, 2, 6, 3, 4, 5))

    out = _assemble(xr, gm, txr, tyr)
    return out.reshape(N_BATCH * NUM_PANEL, N_FEAT, N_WITH_PAD, N_WITH_PAD)
